# baseline (speedup 1.0000x reference)
import functools

import numpy as np
import jax
import jax.numpy as jnp
from jax import lax
from jax.experimental import pallas as pl
from jax.experimental.pallas import tpu as pltpu
from jax.experimental.pallas import tpu_sc as plsc

D_MODEL = 768
N_VOCAB = 100000
SEQ = 4096
BATCH = 4
N_ROWS = SEQ * BATCH
SCALE = float(np.sqrt(np.float32(D_MODEL)))

NC, NS = 2, 16
NW = NC * NS
B_PER_W = N_ROWS // NW
CHUNK = 128
N_CHUNKS = B_PER_W // CHUNK
POS_PER_CHUNK = CHUNK // BATCH
LANES = 16
N_VEC = D_MODEL // LANES


def _positional_encoding() -> np.ndarray:
    position = np.arange(0, SEQ, dtype=np.float32)[:, None]
    two_i = np.arange(0, D_MODEL, 2, dtype=np.float32)
    div_term = np.exp(two_i * -(np.log(10000.0) / D_MODEL))
    enc = np.zeros((SEQ, D_MODEL), dtype=np.float32)
    enc[:, 0::2] = np.sin(position * div_term)
    enc[:, 1::2] = np.cos(position * div_term)
    return enc


_PE_NP = _positional_encoding()


@functools.partial(
    pl.kernel,
    out_type=jax.ShapeDtypeStruct((N_ROWS, D_MODEL), jnp.float32),
    mesh=plsc.VectorSubcoreMesh(core_axis_name="c", subcore_axis_name="s"),
    scratch_types=[
        pltpu.VMEM((N_CHUNKS, CHUNK), jnp.int32),
        pltpu.VMEM((CHUNK, D_MODEL), jnp.float32),
        pltpu.VMEM((POS_PER_CHUNK, D_MODEL), jnp.float32),
        pltpu.SemaphoreType.DMA,
    ],
)
def _emb_pe_kernel(x_hbm, w_hbm, pe_hbm, out_hbm, idx_v, emb_v, pe_v, sem):
    wid = lax.axis_index("s") * NC + lax.axis_index("c")
    pltpu.sync_copy(x_hbm.at[wid], idx_v)

    for c in range(N_CHUNKS):
        row0 = wid * B_PER_W + c * CHUNK
        gather = pltpu.async_copy(w_hbm.at[idx_v.at[c]], emb_v, sem)
        pos0 = wid * (B_PER_W // BATCH) + c * POS_PER_CHUNK
        pltpu.sync_copy(pe_hbm.at[pl.ds(pos0, POS_PER_CHUNK)], pe_v)
        gather.wait()

        def pos_body(p, _):
            def col_body(j, _):
                off = j * LANES
                pe_vec = pe_v[p, pl.ds(off, LANES)]
                for b in range(BATCH):
                    r = p * BATCH + b
                    emb_v[r, pl.ds(off, LANES)] = (
                        emb_v[r, pl.ds(off, LANES)] * SCALE + pe_vec
                    )
                return 0

            return lax.fori_loop(0, N_VEC, col_body, 0)

        lax.fori_loop(0, POS_PER_CHUNK, pos_body, 0)
        pltpu.sync_copy(emb_v, out_hbm.at[pl.ds(row0, CHUNK)])


def kernel(x, W):
    xf = x.astype(jnp.int32).reshape(NW, N_CHUNKS, CHUNK)
    pe = jnp.asarray(_PE_NP)
    out = _emb_pe_kernel(xf, W, pe)
    return out.reshape(SEQ, BATCH, D_MODEL)
